# baseline (speedup 1.0000x reference)
import functools
import math

import jax
import jax.numpy as jnp
import numpy as np
from jax import lax
from jax.experimental import pallas as pl
from jax.experimental.pallas import tpu as pltpu
from jax.experimental.pallas import tpu_sc as plsc

N_CH = 256
IN_BS = 49152
OUT_BS = math.ceil(IN_BS * 44100 / 48000)

NW = 32
LANES = 16
JW = 1408
JBUF = 1536
J_PAD = NW * JW + (JBUF - JW)
NVEC_STD = JW // LANES
NVEC_LAST = JBUF // LANES
WIN = 1664
CB = 8
NCB = N_CH // CB


def _start(w):
    return 128 * ((49039 * w) >> 12)


def _host_tables():
    sf = np.float32((IN_BS - 1) / (OUT_BS - 1) + 1e-12)
    jg = np.arange(OUT_BS, dtype=np.float32)
    xg = jg * sf
    y0g = np.floor(xg).astype(np.int64)
    y1g = np.clip(y0g + 1, 0, IN_BS - 1)
    fr = np.clip(xg - y0g.astype(np.float32), np.float32(0.0), np.float32(1.0))
    fr[0] = np.float32(0.0)
    fr[-1] = np.round(fr[-1])
    ym1g = np.clip(y0g - 1, 0, IN_BS - 1)
    y2g = np.clip(y1g + 1, 0, IN_BS - 1)

    i0 = np.zeros(NW * JBUF, np.int32)
    i1 = np.zeros(NW * JBUF, np.int32)
    xw = np.zeros(NW * JBUF, np.float32)
    for w in range(NW):
        s = _start(w)
        assert s >= 0 and s + WIN <= IN_BS
        nj = (NVEC_LAST if w == NW - 1 else NVEC_STD) * LANES
        j = w * JW + np.arange(nj)
        real = j < OUT_BS
        jr = j[real]
        loc0 = y0g[jr] - s
        loc1 = y1g[jr] - s
        assert loc0.min() >= (1 if w > 0 else 0) and loc0.max() <= WIN - 1
        assert loc1.min() >= 0 and loc1.max() <= WIN - 1
        assert np.array_equal(np.maximum(loc0 - 1, 0) + s, ym1g[jr])
        assert np.array_equal(np.minimum(loc1 + 1, WIN - 1) + s, y2g[jr])
        blk0 = np.full(nj, 1, np.int32)
        blk0[real] = loc0
        blk1 = np.full(nj, 2, np.int32)
        blk1[real] = loc1
        blkx = np.zeros(nj, np.float32)
        blkx[real] = fr[jr]
        i0[w * JBUF: w * JBUF + nj] = blk0
        i1[w * JBUF: w * JBUF + nj] = blk1
        xw[w * JBUF: w * JBUF + nj] = blkx
    return xw, i0, i1


_XW, _I0, _I1 = _host_tables()


def _resample_body(y_hbm, xw_hbm, i0_hbm, i1_hbm, out_hbm,
                   in0, in1, ob0, ob1, i0_v, i1_v, xw_v,
                   is0, is1, os0, os1):
    wid = lax.axis_index("s") * 2 + lax.axis_index("c")
    j0 = wid * JW
    tb = wid * JBUF
    s_w = 128 * ((wid * 49039) >> 12)
    n_vec = jnp.where(wid == NW - 1, NVEC_LAST, NVEC_STD)

    pltpu.sync_copy(i0_hbm.at[pl.ds(tb, JBUF)], i0_v)
    pltpu.sync_copy(i1_hbm.at[pl.ds(tb, JBUF)], i1_v)
    pltpu.sync_copy(xw_hbm.at[pl.ds(tb, JBUF)], xw_v)

    def start_in(cb, in_b, is_):
        for ch in range(CB):
            pltpu.async_copy(
                y_hbm.at[pl.ds((cb * CB + ch) * IN_BS + s_w, WIN)],
                in_b.at[pl.ds(ch * WIN, WIN)], is_)

    start_in(0, in0, is0)
    start_in(1, in1, is1)

    def compute(in_b, ob):
        for ch in range(CB):
            src = in_b.at[pl.ds(ch * WIN, WIN)]
            base = ch * JBUF

            @plsc.parallel_loop(0, n_vec * LANES, LANES, unroll=6)
            def _(j):
                sl = pl.ds(j, LANES)
                i0 = i0_v[sl]
                i1 = i1_v[sl]
                x = xw_v[sl]
                im1 = jnp.maximum(i0 - 1, 0)
                i2 = jnp.minimum(i1 + 1, WIN - 1)
                ym1 = plsc.load_gather(src, [im1])
                y0 = plsc.load_gather(src, [i0])
                y1 = plsc.load_gather(src, [i1])
                y2 = plsc.load_gather(src, [i2])
                c1 = 0.5 * (y1 - ym1)
                c2 = ym1 - 2.5 * y0 + 2.0 * y1 - 0.5 * y2
                c3 = 1.5 * (y0 - y1) + 0.5 * (y2 - ym1)
                ob[pl.ds(base + j, LANES)] = ((c3 * x + c2) * x + c1) * x + y0

    def wait_in(in_b, is_):
        pltpu.make_async_copy(
            y_hbm.at[pl.ds(0, CB * WIN)], in_b, is_).wait()

    def start_out(cb, ob, os_):
        c0 = cb * CB

        @pl.when(wid == NW - 1)
        def _():
            for ch in range(CB):
                pltpu.async_copy(
                    ob.at[pl.ds(ch * JBUF, JBUF)],
                    out_hbm.at[pl.ds((c0 + ch) * J_PAD + j0, JBUF)], os_)

        @pl.when(wid != NW - 1)
        def _():
            for ch in range(CB):
                pltpu.async_copy(
                    ob.at[pl.ds(ch * JBUF, JW)],
                    out_hbm.at[pl.ds((c0 + ch) * J_PAD + j0, JW)], os_)

    def wait_out(ob, os_):
        @pl.when(wid == NW - 1)
        def _():
            pltpu.make_async_copy(
                out_hbm.at[pl.ds(0, CB * JBUF)], ob, os_).wait()

        @pl.when(wid != NW - 1)
        def _():
            pltpu.make_async_copy(
                out_hbm.at[pl.ds(0, CB * JW)],
                ob.at[pl.ds(0, CB * JW)], os_).wait()

    def phase(t, cb, in_b, ob, is_, os_):
        wait_in(in_b, is_)

        @pl.when(t > 0)
        def _():
            wait_out(ob, os_)

        compute(in_b, ob)
        start_out(cb, ob, os_)

        @pl.when(cb + 2 < NCB)
        def _():
            start_in(cb + 2, in_b, is_)

    def tbody(t, _):
        phase(t, 2 * t, in0, ob0, is0, os0)
        phase(t, 2 * t + 1, in1, ob1, is1, os1)
        return 0

    lax.fori_loop(0, NCB // 2, tbody, 0)
    wait_out(ob0, os0)
    wait_out(ob1, os1)


@functools.lru_cache(maxsize=1)
def _build():
    mesh = plsc.VectorSubcoreMesh(
        core_axis_name="c", subcore_axis_name="s",
        num_cores=2, num_subcores=16)
    return pl.kernel(
        _resample_body,
        out_type=jax.ShapeDtypeStruct((N_CH * J_PAD,), jnp.float32),
        mesh=mesh,
        compiler_params=pltpu.CompilerParams(needs_layout_passes=False),
        scratch_types=[
            pltpu.VMEM((CB * WIN,), jnp.float32),
            pltpu.VMEM((CB * WIN,), jnp.float32),
            pltpu.VMEM((CB * JBUF,), jnp.float32),
            pltpu.VMEM((CB * JBUF,), jnp.float32),
            pltpu.VMEM((JBUF,), jnp.int32),
            pltpu.VMEM((JBUF,), jnp.int32),
            pltpu.VMEM((JBUF,), jnp.float32),
            pltpu.SemaphoreType.DMA,
            pltpu.SemaphoreType.DMA,
            pltpu.SemaphoreType.DMA,
            pltpu.SemaphoreType.DMA,
        ],
    )


def kernel(y):
    out = _build()(y.reshape(-1), jnp.asarray(_XW), jnp.asarray(_I0),
                   jnp.asarray(_I1))
    return out.reshape(N_CH, J_PAD)[:, :OUT_BS]


# speedup vs baseline: 1.0993x; 1.0993x over previous
import functools
import math

import jax
import jax.numpy as jnp
import numpy as np
from jax import lax
from jax.experimental import pallas as pl
from jax.experimental.pallas import tpu as pltpu
from jax.experimental.pallas import tpu_sc as plsc

N_CH = 256
IN_BS = 49152
OUT_BS = math.ceil(IN_BS * 44100 / 48000)

NW = 32
LANES = 16
JW = 1408
JBUF = 1536
J_PAD = NW * JW + (JBUF - JW)
NVEC_STD = JW // LANES
NVEC_LAST = JBUF // LANES
WIN = 1664
CB = 8
NCB = N_CH // CB


def _start(w):
    return 128 * ((49039 * w) >> 12)


def _host_tables():
    sf = np.float32((IN_BS - 1) / (OUT_BS - 1) + 1e-12)
    jg = np.arange(OUT_BS, dtype=np.float32)
    xg = jg * sf
    y0g = np.floor(xg).astype(np.int64)
    y1g = np.clip(y0g + 1, 0, IN_BS - 1)
    fr = np.clip(xg - y0g.astype(np.float32), np.float32(0.0), np.float32(1.0))
    fr[0] = np.float32(0.0)
    fr[-1] = np.round(fr[-1])
    ym1g = np.clip(y0g - 1, 0, IN_BS - 1)
    y2g = np.clip(y1g + 1, 0, IN_BS - 1)

    i0 = np.zeros(NW * JBUF, np.int32)
    i1 = np.zeros(NW * JBUF, np.int32)
    xw = np.zeros(NW * JBUF, np.float32)
    for w in range(NW):
        s = _start(w)
        assert s >= 0 and s + WIN <= IN_BS
        nj = (NVEC_LAST if w == NW - 1 else NVEC_STD) * LANES
        j = w * JW + np.arange(nj)
        real = j < OUT_BS
        jr = j[real]
        loc0 = y0g[jr] - s
        loc1 = y1g[jr] - s
        assert loc0.min() >= (1 if w > 0 else 0) and loc0.max() <= WIN - 1
        assert loc1.min() >= 0 and loc1.max() <= WIN - 1
        assert np.array_equal(np.maximum(loc0 - 1, 0) + s, ym1g[jr])
        assert np.array_equal(np.minimum(loc1 + 1, WIN - 1) + s, y2g[jr])
        blk0 = np.full(nj, 1, np.int32)
        blk0[real] = loc0
        blk1 = np.full(nj, 2, np.int32)
        blk1[real] = loc1
        blkx = np.zeros(nj, np.float32)
        blkx[real] = fr[jr]
        i0[w * JBUF: w * JBUF + nj] = blk0
        i1[w * JBUF: w * JBUF + nj] = blk1
        xw[w * JBUF: w * JBUF + nj] = blkx
    return xw, i0, i1


_XW, _I0, _I1 = _host_tables()


def _resample_body(y_hbm, xw_hbm, i0_hbm, i1_hbm, out_hbm,
                   in0, in1, ob0, ob1, i0_v, i1_v, xw_v,
                   is0, is1, os0, os1):
    wid = lax.axis_index("s") * 2 + lax.axis_index("c")
    j0 = wid * JW
    tb = wid * JBUF
    s_w = 128 * ((wid * 49039) >> 12)
    n_vec = jnp.where(wid == NW - 1, NVEC_LAST, NVEC_STD)

    pltpu.sync_copy(i0_hbm.at[pl.ds(tb, JBUF)], i0_v)
    pltpu.sync_copy(i1_hbm.at[pl.ds(tb, JBUF)], i1_v)
    pltpu.sync_copy(xw_hbm.at[pl.ds(tb, JBUF)], xw_v)

    def start_in(cb, in_b, is_):
        for ch in range(CB):
            pltpu.async_copy(
                y_hbm.at[pl.ds((cb * CB + ch) * IN_BS + s_w, WIN)],
                in_b.at[pl.ds(ch * WIN, WIN)], is_)

    start_in(0, in0, is0)
    start_in(1, in1, is1)

    def compute(in_b, ob):
        for ch in range(CB):
            src = in_b.at[pl.ds(ch * WIN, WIN)]
            base = ch * JBUF

            @plsc.parallel_loop(0, n_vec * LANES, LANES, unroll=4)
            def _(j):
                sl = pl.ds(j, LANES)
                i0 = i0_v[sl]
                i1 = i1_v[sl]
                x = xw_v[sl]
                im1 = jnp.maximum(i0 - 1, 0)
                i2 = jnp.minimum(i1 + 1, WIN - 1)
                ym1 = plsc.load_gather(src, [im1])
                y0 = plsc.load_gather(src, [i0])
                y1 = plsc.load_gather(src, [i1])
                y2 = plsc.load_gather(src, [i2])
                c1 = 0.5 * (y1 - ym1)
                c2 = ym1 - 2.5 * y0 + 2.0 * y1 - 0.5 * y2
                c3 = 1.5 * (y0 - y1) + 0.5 * (y2 - ym1)
                ob[pl.ds(base + j, LANES)] = ((c3 * x + c2) * x + c1) * x + y0

    def wait_in(in_b, is_):
        pltpu.make_async_copy(
            y_hbm.at[pl.ds(0, CB * WIN)], in_b, is_).wait()

    def start_out(cb, ob, os_):
        c0 = cb * CB

        @pl.when(wid == NW - 1)
        def _():
            for ch in range(CB):
                pltpu.async_copy(
                    ob.at[pl.ds(ch * JBUF, JBUF)],
                    out_hbm.at[pl.ds((c0 + ch) * J_PAD + j0, JBUF)], os_)

        @pl.when(wid != NW - 1)
        def _():
            for ch in range(CB):
                pltpu.async_copy(
                    ob.at[pl.ds(ch * JBUF, JW)],
                    out_hbm.at[pl.ds((c0 + ch) * J_PAD + j0, JW)], os_)

    def wait_out(ob, os_):
        @pl.when(wid == NW - 1)
        def _():
            pltpu.make_async_copy(
                out_hbm.at[pl.ds(0, CB * JBUF)], ob, os_).wait()

        @pl.when(wid != NW - 1)
        def _():
            pltpu.make_async_copy(
                out_hbm.at[pl.ds(0, CB * JW)],
                ob.at[pl.ds(0, CB * JW)], os_).wait()

    def phase(t, cb, in_b, ob, is_, os_):
        wait_in(in_b, is_)

        @pl.when(t > 0)
        def _():
            wait_out(ob, os_)

        compute(in_b, ob)
        start_out(cb, ob, os_)

        @pl.when(cb + 2 < NCB)
        def _():
            start_in(cb + 2, in_b, is_)

    def tbody(t, _):
        phase(t, 2 * t, in0, ob0, is0, os0)
        phase(t, 2 * t + 1, in1, ob1, is1, os1)
        return 0

    lax.fori_loop(0, NCB // 2, tbody, 0)
    wait_out(ob0, os0)
    wait_out(ob1, os1)


@functools.lru_cache(maxsize=1)
def _build():
    mesh = plsc.VectorSubcoreMesh(
        core_axis_name="c", subcore_axis_name="s",
        num_cores=2, num_subcores=16)
    return pl.kernel(
        _resample_body,
        out_type=jax.ShapeDtypeStruct((N_CH * J_PAD,), jnp.float32),
        mesh=mesh,
        compiler_params=pltpu.CompilerParams(needs_layout_passes=False),
        scratch_types=[
            pltpu.VMEM((CB * WIN,), jnp.float32),
            pltpu.VMEM((CB * WIN,), jnp.float32),
            pltpu.VMEM((CB * JBUF,), jnp.float32),
            pltpu.VMEM((CB * JBUF,), jnp.float32),
            pltpu.VMEM((JBUF,), jnp.int32),
            pltpu.VMEM((JBUF,), jnp.int32),
            pltpu.VMEM((JBUF,), jnp.float32),
            pltpu.SemaphoreType.DMA,
            pltpu.SemaphoreType.DMA,
            pltpu.SemaphoreType.DMA,
            pltpu.SemaphoreType.DMA,
        ],
    )


def kernel(y):
    out = _build()(y.reshape(-1), jnp.asarray(_XW), jnp.asarray(_I0),
                   jnp.asarray(_I1))
    return out.reshape(N_CH, J_PAD)[:, :OUT_BS]
